# baseline (speedup 1.0000x reference)
import functools

import jax
import jax.numpy as jnp
from jax import lax
from jax.experimental import pallas as pl
from jax.experimental.pallas import tpu as pltpu
from jax.experimental.pallas import tpu_sc as plsc

N = 10000
NPAD = 10112
EMB = 64
W128 = 128
T = 24
E = 320000
P = 20000
NTILES = 32
ECHUNK = 128
ECHUNKS = 80
EPAD = NTILES * ECHUNKS * ECHUNK
PCHUNKS = 5
PPAD = NTILES * PCHUNKS * ECHUNK
ROWS_PER_TILE = NPAD // 16
F32 = jnp.float32
I32 = jnp.int32
HI = lax.Precision.HIGHEST


def _mesh():
    return plsc.VectorSubcoreMesh(core_axis_name="c", subcore_axis_name="s",
                                  num_cores=2, num_subcores=16)


def _zero_fill(buf, d):
    zero = jnp.zeros((16,), F32)

    def zrow(r, carry):
        for cb in range(d // 16):
            buf[r, pl.ds(cb * 16, 16)] = zero
        return carry

    lax.fori_loop(0, ECHUNK, zrow, 0)


def _tile_rows(row0):
    out = []
    done = 0
    while done < ROWS_PER_TILE:
        n = min(128, ROWS_PER_TILE - done)
        out.append((row0 + done, n))
        done += n
    return out


def _make_seg_sum(d, stage_z):
  scratch = [
        pltpu.VMEM_SHARED((NPAD, d), F32),
        pltpu.VMEM((ECHUNKS, ECHUNK), I32),
        pltpu.VMEM((2, ECHUNK), I32),
        pltpu.VMEM((2, ECHUNK), I32),
        pltpu.VMEM((ECHUNK, d), F32),
        pltpu.VMEM((ECHUNK, d), F32),
        pltpu.SemaphoreType.DMA,
        pltpu.SemaphoreType.DMA,
        pltpu.SemaphoreType.DMA,
        pltpu.SemaphoreType.DMA,
  ]
  if stage_z:
      scratch.append(pltpu.VMEM_SHARED((N, d), F32))

  @functools.partial(
    pl.kernel,
    out_type=jax.ShapeDtypeStruct((2, NPAD, d), F32),
    mesh=_mesh(),
    compiler_params=pltpu.CompilerParams(use_tc_tiling_on_sc=False),
    scratch_types=scratch,
  )
  def _seg_sum(z_hbm, pk_hbm, out_hbm, acc, pk_v, sidx, didx,
             buf_a, buf_b, gsem_a, gsem_b, ssem_a, ssem_b, *maybe_zsh):
    cid = lax.axis_index("c")
    sid = lax.axis_index("s")
    wid = cid * 16 + sid
    pltpu.sync_copy(pk_hbm.at[wid], pk_v)
    if stage_z:
        z_src = maybe_zsh[0]
        zr0 = sid * (N // 16)
        pltpu.sync_copy(z_hbm.at[pl.ds(zr0, N // 16)],
                        z_src.at[pl.ds(zr0, N // 16)])
    else:
        z_src = z_hbm
    _zero_fill(buf_a, d)
    row0 = sid * ROWS_PER_TILE
    for r0, nr in _tile_rows(row0):
        pltpu.sync_copy(buf_a.at[pl.ds(0, nr)], acc.at[pl.ds(r0, nr)])
    plsc.subcore_barrier()

    def unpack(j, slot):
        for g in range(8):
            pk = pk_v[j, pl.ds(g * 16, 16)]
            sidx[slot, pl.ds(g * 16, 16)] = lax.bitwise_and(pk, 0x3FFF)
            didx[slot, pl.ds(g * 16, 16)] = lax.shift_right_logical(pk, 14)

    def body(j, carry):
        c0 = 2 * j
        c1 = 2 * j + 1
        unpack(c0, 0)
        ga = pltpu.async_copy(z_src.at[sidx.at[0]], buf_a, gsem_a)
        unpack(c1, 1)
        gb = pltpu.async_copy(z_src.at[sidx.at[1]], buf_b, gsem_b)
        ga.wait()
        sa = pltpu.async_copy(buf_a, acc.at[didx.at[0]], ssem_a, add=True)
        gb.wait()
        sa.wait()
        sb = pltpu.async_copy(buf_b, acc.at[didx.at[1]], ssem_b, add=True)
        sb.wait()
        return carry

    lax.fori_loop(0, ECHUNKS // 2, body, 0)
    plsc.subcore_barrier()
    for r0, nr in _tile_rows(row0):
        pltpu.sync_copy(acc.at[pl.ds(r0, nr)], out_hbm.at[cid, pl.ds(r0, nr)])

  return _seg_sum


_seg64 = _make_seg_sum(EMB, stage_z=True)
_seg128 = _make_seg_sum(W128, stage_z=False)


def _make_count(shift):
    @functools.partial(
        pl.kernel,
        out_type=jax.ShapeDtypeStruct((2, NPAD, EMB), F32),
        mesh=_mesh(),
        compiler_params=pltpu.CompilerParams(use_tc_tiling_on_sc=False),
        scratch_types=[
            pltpu.VMEM_SHARED((NPAD, EMB), F32),
            pltpu.VMEM((ECHUNKS, ECHUNK), I32),
            pltpu.VMEM((1, ECHUNK), I32),
            pltpu.VMEM((ECHUNK, EMB), F32),
            pltpu.SemaphoreType.DMA,
        ],
    )
    def count(pk_hbm, out_hbm, acc, pk_v, idx, ones_v, sem):
        cid = lax.axis_index("c")
        sid = lax.axis_index("s")
        wid = cid * 16 + sid
        pltpu.sync_copy(pk_hbm.at[wid], pk_v)
        _zero_fill(ones_v, EMB)
        row0 = sid * ROWS_PER_TILE
        for r0, nr in _tile_rows(row0):
            pltpu.sync_copy(ones_v.at[pl.ds(0, nr)], acc.at[pl.ds(r0, nr)])
        one = jnp.ones((16,), F32)

        def orow(r, carry):
            for cb in range(EMB // 16):
                ones_v[r, pl.ds(cb * 16, 16)] = one
            return carry

        lax.fori_loop(0, ECHUNK, orow, 0)
        plsc.subcore_barrier()

        def body(j, carry):
            for g in range(8):
                pk = pk_v[j, pl.ds(g * 16, 16)]
                idx[0, pl.ds(g * 16, 16)] = lax.bitwise_and(
                    lax.shift_right_logical(pk, shift), 0x3FFF)
            s = pltpu.async_copy(ones_v, acc.at[idx.at[0]], sem, add=True)
            s.wait()
            return carry

        lax.fori_loop(0, ECHUNKS, body, 0)
        plsc.subcore_barrier()
        for r0, nr in _tile_rows(row0):
            pltpu.sync_copy(acc.at[pl.ds(r0, nr)], out_hbm.at[cid, pl.ds(r0, nr)])

    return count


_count_lo = _make_count(0)
_count_hi = _make_count(14)


@functools.partial(
    pl.kernel,
    out_type=(jax.ShapeDtypeStruct((PPAD, W128), F32),
              jax.ShapeDtypeStruct((PPAD, W128), F32),
              jax.ShapeDtypeStruct((PPAD, W128), F32)),
    mesh=_mesh(),
    compiler_params=pltpu.CompilerParams(use_tc_tiling_on_sc=False),
    scratch_types=[
        pltpu.VMEM((PCHUNKS, ECHUNK), I32),
        pltpu.VMEM((PCHUNKS, ECHUNK), I32),
        pltpu.VMEM((PCHUNKS, ECHUNK), I32),
        pltpu.VMEM((ECHUNK, W128), F32),
        pltpu.VMEM((ECHUNK, W128), F32),
        pltpu.VMEM((ECHUNK, W128), F32),
    ],
)
def _pair_gather(comb_hbm, dis_hbm, ori_hbm, dst_hbm,
                 pa_out, pb_out, ds_out, ori_v, dst_v, row_v,
                 buf_a, buf_b, buf_d):
    cid = lax.axis_index("c")
    sid = lax.axis_index("s")
    wid = cid * 16 + sid
    pltpu.sync_copy(ori_hbm.at[wid], ori_v)
    pltpu.sync_copy(dst_hbm.at[wid], dst_v)
    for j in range(PCHUNKS):
        for g in range(8):
            o16 = ori_v[j, pl.ds(g * 16, 16)]
            d16 = dst_v[j, pl.ds(g * 16, 16)]
            fidx = o16 * N + d16
            row_v[j, pl.ds(g * 16, 16)] = lax.shift_right_logical(fidx, 7)
    for j in range(PCHUNKS):
        base = wid * (PCHUNKS * ECHUNK) + j * ECHUNK
        pltpu.sync_copy(comb_hbm.at[ori_v.at[j]], buf_a)
        pltpu.sync_copy(buf_a, pa_out.at[pl.ds(base, ECHUNK)])
        pltpu.sync_copy(comb_hbm.at[dst_v.at[j]], buf_b)
        pltpu.sync_copy(buf_b, pb_out.at[pl.ds(base, ECHUNK)])
        pltpu.sync_copy(dis_hbm.at[row_v.at[j]], buf_d)
        pltpu.sync_copy(buf_d, ds_out.at[pl.ds(base, ECHUNK)])


BR = 2000
GRID = (N // BR,)


def _dot(a, b):
    return jnp.dot(a, b, preferred_element_type=F32, precision=HI)


def _pad128(z):
    return jnp.concatenate(
        [z, jnp.zeros((z.shape[0], W128 - z.shape[1]), F32)], axis=1)


def _row(width):
    return pl.BlockSpec((BR, width), lambda i: (i, 0))


def _pblk(d=EMB):
    return pl.BlockSpec((2, BR, d), lambda i: (0, i, 0))


def _full(shape):
    nd = len(shape)
    return pl.BlockSpec(shape, lambda i: (0,) * nd)


def _tc1_body(h_ref, xt_ref, nrm_ref, wh_ref, wx_ref, o_ref):
    on = nrm_ref[:, 0:1]
    o_ref[...] = (_dot(h_ref[...] * on, wh_ref[...])
                  + _dot(xt_ref[...] * on, wx_ref[...]))


def _tc_mid_body(p_ref, nrm_ref, b_ref, w_ref, o_ref):
    on = nrm_ref[:, 0:1]
    inn = nrm_ref[:, 1:2]
    a = jax.nn.relu(inn * (p_ref[0, :, :] + p_ref[1, :, :]) + b_ref[...])
    o_ref[...] = _dot(a * on, w_ref[...])


def _tc_gate_body(p_ref, nrm_ref, b_ref, h_ref, xt_ref, wh_ref, wx_ref,
                  z4_ref, zg_ref):
    on = nrm_ref[:, 0:1]
    inn = nrm_ref[:, 1:2]
    ru = jax.nn.sigmoid(inn * (p_ref[0, :, :] + p_ref[1, :, :]) + b_ref[...])
    r = ru[:, :EMB]
    zg = ru[:, EMB:]
    z4_ref[...] = (_dot(h_ref[...] * r * on, wh_ref[...])
                   + _dot(xt_ref[...] * on, wx_ref[...]))
    zg_ref[...] = zg


def _tc_final_body(p_ref, nrm_ref, b_ref, zg_ref, h_ref, wab_ref,
                   h2_ref, proj_ref):
    inn = nrm_ref[:, 1:2]
    hn = jnp.tanh(inn * (p_ref[0, :, :] + p_ref[1, :, :]) + b_ref[...])
    zg = zg_ref[...]
    h2 = zg * h_ref[...] + (1.0 - zg) * hn
    h2_ref[...] = h2
    proj_ref[...] = _dot(h2, wab_ref[...])


def _tc_norms_body(dgo_ref, dgi_ref, nrm_ref):
    od = dgo_ref[0, :, 0:1] + dgo_ref[1, :, 0:1]
    idg = dgi_ref[0, :, 0:1] + dgi_ref[1, :, 0:1]
    on = jnp.where(od > 0, lax.rsqrt(jnp.maximum(od, 1.0)), 0.0)
    inn = jnp.where(idg > 0, lax.rsqrt(jnp.maximum(idg, 1.0)), 0.0)
    nrm_ref[...] = jnp.concatenate([on, inn], axis=1)


PBR = 4096


def _tc_predf_body(pa_ref, pb_ref, ds_ref, col_ref, wt_ref, b_ref, o_ref):
    cols = lax.broadcasted_iota(I32, (1, ECHUNK), 1)
    onehot = (cols == col_ref[...]).astype(F32)
    dv = jnp.sum(ds_ref[...] * onehot, axis=1, keepdims=True)
    s = pa_ref[:, :T] + pb_ref[:, T:2 * T] + dv * wt_ref[...] + b_ref[...]
    o_ref[...] = jnp.tanh(s)


def _call(body, out_shape, *args):
    return pl.pallas_call(body, out_shape=out_shape)(*args)


def _gcall(body, out_shape, out_specs, in_specs, *args):
    return pl.pallas_call(body, out_shape=out_shape, grid=GRID,
                          in_specs=in_specs, out_specs=out_specs)(*args)


def _sds(shape):
    return jax.ShapeDtypeStruct(shape, F32)


def kernel(x, dis, edge_index, train_idx, ru_W_in, ru_b_in, ru_W_hid, ru_b_hid,
           ru_W_out, ru_b_out, rh_W_in, rh_b_in, rh_W_hid, rh_b_hid, rh_W_out,
           rh_b_out, pred_W, pred_b):
    src = edge_index[0].astype(I32)
    dst = edge_index[1].astype(I32)
    npadE = EPAD - E
    fill_g = jnp.arange(npadE, dtype=I32) % N
    fill_d = N + jnp.arange(npadE, dtype=I32) % 16
    srcP = jnp.concatenate([src, fill_g])
    dstP = jnp.concatenate([dst, fill_d])
    pk3 = (srcP | (dstP << 14)).reshape(NTILES, ECHUNKS, ECHUNK)
    srcD = jnp.concatenate([src, fill_d])
    pkd3 = (srcD | (dstP << 14)).reshape(NTILES, ECHUNKS, ECHUNK)

    ori = train_idx[0].astype(I32)
    dstp = train_idx[1].astype(I32)
    pfill = jnp.arange(PPAD - P, dtype=I32) % N
    ori3 = jnp.concatenate([ori, pfill]).reshape(NTILES, PCHUNKS, ECHUNK)
    dst3p = jnp.concatenate([dstp, pfill]).reshape(NTILES, PCHUNKS, ECHUNK)

    feat = x[:, :, :EMB]
    ruW_h, ruW_x = ru_W_in[:EMB], ru_W_in[EMB:]
    rhW_h, rhW_x = rh_W_in[:EMB], rh_W_in[EMB:]
    rub1 = ru_b_in.reshape(1, EMB)
    rub2 = ru_b_hid.reshape(1, EMB)
    rub3 = ru_b_out.reshape(1, 2 * EMB)
    rhb1 = rh_b_in.reshape(1, EMB)
    rhb2 = rh_b_hid.reshape(1, EMB)
    rhb3 = rh_b_out.reshape(1, EMB)
    wab = jnp.concatenate([pred_W[:EMB], pred_W[EMB:2 * EMB]], axis=1)

    dego = _count_lo(pkd3)
    degi = _count_hi(pkd3)
    norms = _gcall(_tc_norms_body, _sds((N, 2)), _row(2),
                   [_pblk(), _pblk()], dego, degi)

    def step(h, xt):
        mid_specs = [_pblk(), _row(2), _full((1, EMB)), _full((EMB, EMB))]
        z1 = _gcall(_tc1_body, _sds((N, EMB)), _row(EMB),
                    [_row(EMB), _row(EMB), _row(2),
                     _full((EMB, EMB)), _full((EMB, EMB))],
                    h, xt, norms, ruW_h, ruW_x)
        p = _seg64(z1, pk3)
        z2 = _gcall(_tc_mid_body, _sds((N, EMB)), _row(EMB), mid_specs,
                    p, norms, rub1, ru_W_hid)
        p = _seg64(z2, pk3)
        z3 = _gcall(_tc_mid_body, _sds((N, W128)), _row(W128),
                    [_pblk(), _row(2), _full((1, EMB)), _full((EMB, W128))],
                    p, norms, rub2, ru_W_out)
        p = _seg128(z3, pk3)
        z4, zg = _gcall(_tc_gate_body, (_sds((N, EMB)), _sds((N, EMB))),
                        (_row(EMB), _row(EMB)),
                        [_pblk(W128), _row(2), _full((1, W128)), _row(EMB),
                         _row(EMB), _full((EMB, EMB)), _full((EMB, EMB))],
                        p, norms, rub3, h, xt, rhW_h, rhW_x)
        p = _seg64(z4, pk3)
        z5 = _gcall(_tc_mid_body, _sds((N, EMB)), _row(EMB), mid_specs,
                    p, norms, rhb1, rh_W_hid)
        p = _seg64(z5, pk3)
        z6 = _gcall(_tc_mid_body, _sds((N, EMB)), _row(EMB), mid_specs,
                    p, norms, rhb2, rh_W_out)
        p = _seg64(z6, pk3)
        h2, proj = _gcall(_tc_final_body, (_sds((N, EMB)), _sds((N, 2))),
                          (_row(EMB), _row(2)),
                          [_pblk(), _row(2), _full((1, EMB)), _row(EMB),
                           _row(EMB), _full((EMB, 2))],
                          p, norms, rhb3, zg, h, wab)
        return h2, proj

    h0 = jnp.zeros((N, EMB), F32)
    _, projs = lax.scan(step, h0, feat)

    paT = projs[:, :, 0].T
    pbT = projs[:, :, 1].T
    comb = jnp.pad(jnp.concatenate([paT, pbT], axis=1),
                   ((0, 0), (0, W128 - 2 * T)))
    pa_g, pb_g, dslab = _pair_gather(comb, dis.reshape(N * N // 128, 128),
                                     ori3, dst3p)
    oriP = jnp.concatenate([ori, pfill])
    dstP = jnp.concatenate([dstp, pfill])
    colid = ((oriP * N + dstP) % 128).astype(I32).reshape(PPAD, 1)
    pr = lambda w: pl.BlockSpec((PBR, w), lambda i: (i, 0))
    out = pl.pallas_call(
        _tc_predf_body, out_shape=_sds((PPAD, T)), grid=(PPAD // PBR,),
        in_specs=[pr(W128), pr(W128), pr(W128), pr(1),
                  _full((1, 1)), _full((1, 1))],
        out_specs=pr(T),
    )(pa_g, pb_g, dslab, colid, pred_W[2 * EMB:].reshape(1, 1),
      pred_b.reshape(1, 1))
    return out[:P]


# speedup vs baseline: 1.3275x; 1.3275x over previous
import functools

import jax
import jax.numpy as jnp
from jax import lax
from jax.experimental import pallas as pl
from jax.experimental.pallas import tpu as pltpu
from jax.experimental.pallas import tpu_sc as plsc

N = 10000
NPAD = 10112
EMB = 64
W128 = 128
T = 24
E = 320000
P = 20000
NTILES = 32
ECHUNK = 128
ECHUNKS = 80
EPAD = NTILES * ECHUNKS * ECHUNK
PCHUNKS = 5
PPAD = NTILES * PCHUNKS * ECHUNK
ROWS_PER_TILE = NPAD // 16
F32 = jnp.float32
I32 = jnp.int32
HI = lax.Precision.HIGHEST


def _mesh():
    return plsc.VectorSubcoreMesh(core_axis_name="c", subcore_axis_name="s",
                                  num_cores=2, num_subcores=16)


def _zero_fill(buf, d):
    zero = jnp.zeros((16,), F32)

    def zrow(r, carry):
        for cb in range(d // 16):
            buf[r, pl.ds(cb * 16, 16)] = zero
        return carry

    lax.fori_loop(0, ECHUNK, zrow, 0)


def _tile_rows(row0):
    out = []
    done = 0
    while done < ROWS_PER_TILE:
        n = min(128, ROWS_PER_TILE - done)
        out.append((row0 + done, n))
        done += n
    return out


def _make_seg_sum(d, nbuf):
  scratch = [
        pltpu.VMEM_SHARED((NPAD, d), F32),
        pltpu.VMEM((ECHUNKS, ECHUNK), I32),
        pltpu.VMEM((nbuf, ECHUNK), I32),
        pltpu.VMEM((nbuf, ECHUNK), I32),
  ]
  scratch += [pltpu.VMEM((ECHUNK, d), F32)] * nbuf
  scratch += [pltpu.SemaphoreType.DMA] * (2 * nbuf)

  @functools.partial(
    pl.kernel,
    out_type=jax.ShapeDtypeStruct((2, NPAD, d), F32),
    mesh=_mesh(),
    compiler_params=pltpu.CompilerParams(use_tc_tiling_on_sc=False),
    scratch_types=scratch,
  )
  def _seg_sum(z_hbm, pk_hbm, out_hbm, acc, pk_v, sidx, didx, *bufs_sems):
    bufs = bufs_sems[:nbuf]
    gsems = bufs_sems[nbuf:2 * nbuf]
    ssems = bufs_sems[2 * nbuf:3 * nbuf]
    cid = lax.axis_index("c")
    sid = lax.axis_index("s")
    wid = cid * 16 + sid
    pltpu.sync_copy(pk_hbm.at[wid], pk_v)
    _zero_fill(bufs[0], d)
    row0 = sid * ROWS_PER_TILE
    for r0, nr in _tile_rows(row0):
        pltpu.sync_copy(bufs[0].at[pl.ds(0, nr)], acc.at[pl.ds(r0, nr)])
    plsc.subcore_barrier()

    def unpack(c, slot):
        for g in range(8):
            pk = pk_v[c, pl.ds(g * 16, 16)]
            sidx[slot, pl.ds(g * 16, 16)] = lax.bitwise_and(pk, 0x3FFF)
            didx[slot, pl.ds(g * 16, 16)] = lax.shift_right_logical(pk, 14)

    def gather(k):
        pltpu.async_copy(z_hbm.at[sidx.at[k]], bufs[k], gsems[k])

    def gather_wait(k):
        pltpu.make_async_copy(z_hbm.at[sidx.at[k]], bufs[k], gsems[k]).wait()

    def scat(k):
        pltpu.async_copy(bufs[k], acc.at[didx.at[k]], ssems[k], add=True)

    def scat_wait(k):
        pltpu.make_async_copy(bufs[k], acc.at[didx.at[k]], ssems[k]).wait()

    for k in range(nbuf):
        unpack(k, k)
        gather(k)

    nloop = ECHUNKS // nbuf
    def body(j, carry):
        for k in range(nbuf):
            gather_wait(k)
            scat(k)
        for k in range(nbuf):
            c = nbuf * j + k + nbuf
            scat_wait(k)

            @pl.when(j < nloop - 1)
            def _():
                unpack(c, k)
                gather(k)
        return carry

    lax.fori_loop(0, nloop, body, 0)
    plsc.subcore_barrier()
    for r0, nr in _tile_rows(row0):
        pltpu.sync_copy(acc.at[pl.ds(r0, nr)], out_hbm.at[cid, pl.ds(r0, nr)])

  return _seg_sum


_seg64 = _make_seg_sum(EMB, nbuf=4)
_seg128 = _make_seg_sum(W128, nbuf=2)


def _make_count(shift):
    @functools.partial(
        pl.kernel,
        out_type=jax.ShapeDtypeStruct((2, NPAD, EMB), F32),
        mesh=_mesh(),
        compiler_params=pltpu.CompilerParams(use_tc_tiling_on_sc=False),
        scratch_types=[
            pltpu.VMEM_SHARED((NPAD, EMB), F32),
            pltpu.VMEM((ECHUNKS, ECHUNK), I32),
            pltpu.VMEM((1, ECHUNK), I32),
            pltpu.VMEM((ECHUNK, EMB), F32),
            pltpu.SemaphoreType.DMA,
        ],
    )
    def count(pk_hbm, out_hbm, acc, pk_v, idx, ones_v, sem):
        cid = lax.axis_index("c")
        sid = lax.axis_index("s")
        wid = cid * 16 + sid
        pltpu.sync_copy(pk_hbm.at[wid], pk_v)
        _zero_fill(ones_v, EMB)
        row0 = sid * ROWS_PER_TILE
        for r0, nr in _tile_rows(row0):
            pltpu.sync_copy(ones_v.at[pl.ds(0, nr)], acc.at[pl.ds(r0, nr)])
        one = jnp.ones((16,), F32)

        def orow(r, carry):
            for cb in range(EMB // 16):
                ones_v[r, pl.ds(cb * 16, 16)] = one
            return carry

        lax.fori_loop(0, ECHUNK, orow, 0)
        plsc.subcore_barrier()

        def body(j, carry):
            for g in range(8):
                pk = pk_v[j, pl.ds(g * 16, 16)]
                idx[0, pl.ds(g * 16, 16)] = lax.bitwise_and(
                    lax.shift_right_logical(pk, shift), 0x3FFF)
            s = pltpu.async_copy(ones_v, acc.at[idx.at[0]], sem, add=True)
            s.wait()
            return carry

        lax.fori_loop(0, ECHUNKS, body, 0)
        plsc.subcore_barrier()
        for r0, nr in _tile_rows(row0):
            pltpu.sync_copy(acc.at[pl.ds(r0, nr)], out_hbm.at[cid, pl.ds(r0, nr)])

    return count


_count_lo = _make_count(0)
_count_hi = _make_count(14)


@functools.partial(
    pl.kernel,
    out_type=(jax.ShapeDtypeStruct((PPAD, W128), F32),
              jax.ShapeDtypeStruct((PPAD, W128), F32),
              jax.ShapeDtypeStruct((PPAD, W128), F32)),
    mesh=_mesh(),
    compiler_params=pltpu.CompilerParams(use_tc_tiling_on_sc=False),
    scratch_types=[
        pltpu.VMEM((PCHUNKS, ECHUNK), I32),
        pltpu.VMEM((PCHUNKS, ECHUNK), I32),
        pltpu.VMEM((PCHUNKS, ECHUNK), I32),
        pltpu.VMEM((ECHUNK, W128), F32),
        pltpu.VMEM((ECHUNK, W128), F32),
        pltpu.VMEM((ECHUNK, W128), F32),
    ],
)
def _pair_gather(comb_hbm, dis_hbm, ori_hbm, dst_hbm,
                 pa_out, pb_out, ds_out, ori_v, dst_v, row_v,
                 buf_a, buf_b, buf_d):
    cid = lax.axis_index("c")
    sid = lax.axis_index("s")
    wid = cid * 16 + sid
    pltpu.sync_copy(ori_hbm.at[wid], ori_v)
    pltpu.sync_copy(dst_hbm.at[wid], dst_v)
    for j in range(PCHUNKS):
        for g in range(8):
            o16 = ori_v[j, pl.ds(g * 16, 16)]
            d16 = dst_v[j, pl.ds(g * 16, 16)]
            fidx = o16 * N + d16
            row_v[j, pl.ds(g * 16, 16)] = lax.shift_right_logical(fidx, 7)
    for j in range(PCHUNKS):
        base = wid * (PCHUNKS * ECHUNK) + j * ECHUNK
        pltpu.sync_copy(comb_hbm.at[ori_v.at[j]], buf_a)
        pltpu.sync_copy(buf_a, pa_out.at[pl.ds(base, ECHUNK)])
        pltpu.sync_copy(comb_hbm.at[dst_v.at[j]], buf_b)
        pltpu.sync_copy(buf_b, pb_out.at[pl.ds(base, ECHUNK)])
        pltpu.sync_copy(dis_hbm.at[row_v.at[j]], buf_d)
        pltpu.sync_copy(buf_d, ds_out.at[pl.ds(base, ECHUNK)])


BR = 2000
GRID = (N // BR,)


def _dot(a, b):
    return jnp.dot(a, b, preferred_element_type=F32, precision=HI)


def _pad128(z):
    return jnp.concatenate(
        [z, jnp.zeros((z.shape[0], W128 - z.shape[1]), F32)], axis=1)


def _row(width):
    return pl.BlockSpec((BR, width), lambda i: (i, 0))


def _pblk(d=EMB):
    return pl.BlockSpec((2, BR, d), lambda i: (0, i, 0))


def _full(shape):
    nd = len(shape)
    return pl.BlockSpec(shape, lambda i: (0,) * nd)


def _tc1_body(h_ref, xt_ref, nrm_ref, wh_ref, wx_ref, o_ref):
    on = nrm_ref[:, 0:1]
    o_ref[...] = (_dot(h_ref[...] * on, wh_ref[...])
                  + _dot(xt_ref[...] * on, wx_ref[...]))


def _tc_mid_body(p_ref, nrm_ref, b_ref, w_ref, o_ref):
    on = nrm_ref[:, 0:1]
    inn = nrm_ref[:, 1:2]
    a = jax.nn.relu(inn * (p_ref[0, :, :] + p_ref[1, :, :]) + b_ref[...])
    o_ref[...] = _dot(a * on, w_ref[...])


def _tc_gate_body(p_ref, nrm_ref, b_ref, h_ref, xt_ref, wh_ref, wx_ref,
                  z4_ref, zg_ref):
    on = nrm_ref[:, 0:1]
    inn = nrm_ref[:, 1:2]
    ru = jax.nn.sigmoid(inn * (p_ref[0, :, :] + p_ref[1, :, :]) + b_ref[...])
    r = ru[:, :EMB]
    zg = ru[:, EMB:]
    z4_ref[...] = (_dot(h_ref[...] * r * on, wh_ref[...])
                   + _dot(xt_ref[...] * on, wx_ref[...]))
    zg_ref[...] = zg


def _tc_final_body(p_ref, nrm_ref, b_ref, zg_ref, h_ref, wab_ref,
                   h2_ref, proj_ref):
    inn = nrm_ref[:, 1:2]
    hn = jnp.tanh(inn * (p_ref[0, :, :] + p_ref[1, :, :]) + b_ref[...])
    zg = zg_ref[...]
    h2 = zg * h_ref[...] + (1.0 - zg) * hn
    h2_ref[...] = h2
    proj_ref[...] = _dot(h2, wab_ref[...])


def _tc_norms_body(dgo_ref, dgi_ref, nrm_ref):
    od = dgo_ref[0, :, 0:1] + dgo_ref[1, :, 0:1]
    idg = dgi_ref[0, :, 0:1] + dgi_ref[1, :, 0:1]
    on = jnp.where(od > 0, lax.rsqrt(jnp.maximum(od, 1.0)), 0.0)
    inn = jnp.where(idg > 0, lax.rsqrt(jnp.maximum(idg, 1.0)), 0.0)
    nrm_ref[...] = jnp.concatenate([on, inn], axis=1)


PBR = 4096


def _tc_predf_body(pa_ref, pb_ref, ds_ref, col_ref, wt_ref, b_ref, o_ref):
    cols = lax.broadcasted_iota(I32, (1, ECHUNK), 1)
    onehot = (cols == col_ref[...]).astype(F32)
    dv = jnp.sum(ds_ref[...] * onehot, axis=1, keepdims=True)
    s = pa_ref[:, :T] + pb_ref[:, T:2 * T] + dv * wt_ref[...] + b_ref[...]
    o_ref[...] = jnp.tanh(s)


def _call(body, out_shape, *args):
    return pl.pallas_call(body, out_shape=out_shape)(*args)


def _gcall(body, out_shape, out_specs, in_specs, *args):
    return pl.pallas_call(body, out_shape=out_shape, grid=GRID,
                          in_specs=in_specs, out_specs=out_specs)(*args)


def _sds(shape):
    return jax.ShapeDtypeStruct(shape, F32)


def kernel(x, dis, edge_index, train_idx, ru_W_in, ru_b_in, ru_W_hid, ru_b_hid,
           ru_W_out, ru_b_out, rh_W_in, rh_b_in, rh_W_hid, rh_b_hid, rh_W_out,
           rh_b_out, pred_W, pred_b):
    src = edge_index[0].astype(I32)
    dst = edge_index[1].astype(I32)
    npadE = EPAD - E
    fill_g = jnp.arange(npadE, dtype=I32) % N
    fill_d = N + jnp.arange(npadE, dtype=I32) % 16
    srcP = jnp.concatenate([src, fill_g])
    dstP = jnp.concatenate([dst, fill_d])
    pk3 = (srcP | (dstP << 14)).reshape(NTILES, ECHUNKS, ECHUNK)
    srcD = jnp.concatenate([src, fill_d])
    pkd3 = (srcD | (dstP << 14)).reshape(NTILES, ECHUNKS, ECHUNK)

    ori = train_idx[0].astype(I32)
    dstp = train_idx[1].astype(I32)
    pfill = jnp.arange(PPAD - P, dtype=I32) % N
    ori3 = jnp.concatenate([ori, pfill]).reshape(NTILES, PCHUNKS, ECHUNK)
    dst3p = jnp.concatenate([dstp, pfill]).reshape(NTILES, PCHUNKS, ECHUNK)

    feat = x[:, :, :EMB]
    ruW_h, ruW_x = ru_W_in[:EMB], ru_W_in[EMB:]
    rhW_h, rhW_x = rh_W_in[:EMB], rh_W_in[EMB:]
    rub1 = ru_b_in.reshape(1, EMB)
    rub2 = ru_b_hid.reshape(1, EMB)
    rub3 = ru_b_out.reshape(1, 2 * EMB)
    rhb1 = rh_b_in.reshape(1, EMB)
    rhb2 = rh_b_hid.reshape(1, EMB)
    rhb3 = rh_b_out.reshape(1, EMB)
    wab = jnp.concatenate([pred_W[:EMB], pred_W[EMB:2 * EMB]], axis=1)

    dego = _count_lo(pkd3)
    degi = _count_hi(pkd3)
    norms = _gcall(_tc_norms_body, _sds((N, 2)), _row(2),
                   [_pblk(), _pblk()], dego, degi)

    def step(h, xt):
        mid_specs = [_pblk(), _row(2), _full((1, EMB)), _full((EMB, EMB))]
        z1 = _gcall(_tc1_body, _sds((N, EMB)), _row(EMB),
                    [_row(EMB), _row(EMB), _row(2),
                     _full((EMB, EMB)), _full((EMB, EMB))],
                    h, xt, norms, ruW_h, ruW_x)
        p = _seg64(z1, pk3)
        z2 = _gcall(_tc_mid_body, _sds((N, EMB)), _row(EMB), mid_specs,
                    p, norms, rub1, ru_W_hid)
        p = _seg64(z2, pk3)
        z3 = _gcall(_tc_mid_body, _sds((N, W128)), _row(W128),
                    [_pblk(), _row(2), _full((1, EMB)), _full((EMB, W128))],
                    p, norms, rub2, ru_W_out)
        p = _seg128(z3, pk3)
        z4, zg = _gcall(_tc_gate_body, (_sds((N, EMB)), _sds((N, EMB))),
                        (_row(EMB), _row(EMB)),
                        [_pblk(W128), _row(2), _full((1, W128)), _row(EMB),
                         _row(EMB), _full((EMB, EMB)), _full((EMB, EMB))],
                        p, norms, rub3, h, xt, rhW_h, rhW_x)
        p = _seg64(z4, pk3)
        z5 = _gcall(_tc_mid_body, _sds((N, EMB)), _row(EMB), mid_specs,
                    p, norms, rhb1, rh_W_hid)
        p = _seg64(z5, pk3)
        z6 = _gcall(_tc_mid_body, _sds((N, EMB)), _row(EMB), mid_specs,
                    p, norms, rhb2, rh_W_out)
        p = _seg64(z6, pk3)
        h2, proj = _gcall(_tc_final_body, (_sds((N, EMB)), _sds((N, 2))),
                          (_row(EMB), _row(2)),
                          [_pblk(), _row(2), _full((1, EMB)), _row(EMB),
                           _row(EMB), _full((EMB, 2))],
                          p, norms, rhb3, zg, h, wab)
        return h2, proj

    h0 = jnp.zeros((N, EMB), F32)
    _, projs = lax.scan(step, h0, feat)

    paT = projs[:, :, 0].T
    pbT = projs[:, :, 1].T
    comb = jnp.pad(jnp.concatenate([paT, pbT], axis=1),
                   ((0, 0), (0, W128 - 2 * T)))
    pa_g, pb_g, dslab = _pair_gather(comb, dis.reshape(N * N // 128, 128),
                                     ori3, dst3p)
    oriP = jnp.concatenate([ori, pfill])
    dstP = jnp.concatenate([dstp, pfill])
    colid = ((oriP * N + dstP) % 128).astype(I32).reshape(PPAD, 1)
    pr = lambda w: pl.BlockSpec((PBR, w), lambda i: (i, 0))
    out = pl.pallas_call(
        _tc_predf_body, out_shape=_sds((PPAD, T)), grid=(PPAD // PBR,),
        in_specs=[pr(W128), pr(W128), pr(W128), pr(1),
                  _full((1, 1)), _full((1, 1))],
        out_specs=pr(T),
    )(pa_g, pb_g, dslab, colid, pred_W[2 * EMB:].reshape(1, 1),
      pred_b.reshape(1, 1))
    return out[:P]


# speedup vs baseline: 1.3665x; 1.0294x over previous
import functools

import jax
import jax.numpy as jnp
from jax import lax
from jax.experimental import pallas as pl
from jax.experimental.pallas import tpu as pltpu
from jax.experimental.pallas import tpu_sc as plsc

N = 10000
NPAD = 10112
EMB = 64
W128 = 128
T = 24
E = 320000
P = 20000
NTILES = 32
ECHUNK = 128
ECHUNKS = 80
EPAD = NTILES * ECHUNKS * ECHUNK
PCHUNKS = 5
PPAD = NTILES * PCHUNKS * ECHUNK
ROWS_PER_TILE = NPAD // 16
F32 = jnp.float32
I32 = jnp.int32
HI = lax.Precision.HIGHEST


def _mesh():
    return plsc.VectorSubcoreMesh(core_axis_name="c", subcore_axis_name="s",
                                  num_cores=2, num_subcores=16)


def _zero_fill(buf, d):
    zero = jnp.zeros((16,), F32)

    def zrow(r, carry):
        for cb in range(d // 16):
            buf[r, pl.ds(cb * 16, 16)] = zero
        return carry

    lax.fori_loop(0, ECHUNK, zrow, 0)


def _tile_rows(row0):
    out = []
    done = 0
    while done < ROWS_PER_TILE:
        n = min(128, ROWS_PER_TILE - done)
        out.append((row0 + done, n))
        done += n
    return out


def _make_seg_sum(d, nbuf):
  scratch = [
        pltpu.VMEM_SHARED((NPAD, d), F32),
        pltpu.VMEM((ECHUNKS, ECHUNK), I32),
        pltpu.VMEM((nbuf, ECHUNK), I32),
        pltpu.VMEM((nbuf, ECHUNK), I32),
  ]
  scratch += [pltpu.VMEM((ECHUNK, d), F32)] * nbuf
  scratch += [pltpu.SemaphoreType.DMA] * (2 * nbuf)

  @functools.partial(
    pl.kernel,
    out_type=jax.ShapeDtypeStruct((2, NPAD, d), F32),
    mesh=_mesh(),
    compiler_params=pltpu.CompilerParams(use_tc_tiling_on_sc=False),
    scratch_types=scratch,
  )
  def _seg_sum(z_hbm, pk_hbm, out_hbm, acc, pk_v, sidx, didx, *bufs_sems):
    bufs = bufs_sems[:nbuf]
    gsems = bufs_sems[nbuf:2 * nbuf]
    ssems = bufs_sems[2 * nbuf:3 * nbuf]
    cid = lax.axis_index("c")
    sid = lax.axis_index("s")
    wid = cid * 16 + sid
    pltpu.sync_copy(pk_hbm.at[wid], pk_v)
    _zero_fill(bufs[0], d)
    row0 = sid * ROWS_PER_TILE
    for r0, nr in _tile_rows(row0):
        pltpu.sync_copy(bufs[0].at[pl.ds(0, nr)], acc.at[pl.ds(r0, nr)])
    plsc.subcore_barrier()

    def unpack(c, slot):
        for g in range(8):
            pk = pk_v[c, pl.ds(g * 16, 16)]
            sidx[slot, pl.ds(g * 16, 16)] = lax.bitwise_and(pk, 0x3FFF)
            didx[slot, pl.ds(g * 16, 16)] = lax.shift_right_logical(pk, 14)

    def gather(k):
        pltpu.async_copy(z_hbm.at[sidx.at[k]], bufs[k], gsems[k])

    def gather_wait(k):
        pltpu.make_async_copy(z_hbm.at[sidx.at[k]], bufs[k], gsems[k]).wait()

    def scat(k):
        pltpu.async_copy(bufs[k], acc.at[didx.at[k]], ssems[k], add=True)

    def scat_wait(k):
        pltpu.make_async_copy(bufs[k], acc.at[didx.at[k]], ssems[k]).wait()

    for k in range(nbuf):
        unpack(k, k)
        gather(k)

    nloop = ECHUNKS // nbuf
    def body(j, carry):
        for k in range(nbuf):
            gather_wait(k)
            scat(k)
        for k in range(nbuf):
            c = nbuf * j + k + nbuf
            scat_wait(k)

            @pl.when(j < nloop - 1)
            def _():
                unpack(c, k)
                gather(k)
        return carry

    lax.fori_loop(0, nloop, body, 0)
    plsc.subcore_barrier()
    for r0, nr in _tile_rows(row0):
        pltpu.sync_copy(acc.at[pl.ds(r0, nr)], out_hbm.at[cid, pl.ds(r0, nr)])

  return _seg_sum


_seg64 = _make_seg_sum(EMB, nbuf=8)
_seg128 = _make_seg_sum(W128, nbuf=2)


def _make_count(shift):
    @functools.partial(
        pl.kernel,
        out_type=jax.ShapeDtypeStruct((2, NPAD, EMB), F32),
        mesh=_mesh(),
        compiler_params=pltpu.CompilerParams(use_tc_tiling_on_sc=False),
        scratch_types=[
            pltpu.VMEM_SHARED((NPAD, EMB), F32),
            pltpu.VMEM((ECHUNKS, ECHUNK), I32),
            pltpu.VMEM((1, ECHUNK), I32),
            pltpu.VMEM((ECHUNK, EMB), F32),
            pltpu.SemaphoreType.DMA,
        ],
    )
    def count(pk_hbm, out_hbm, acc, pk_v, idx, ones_v, sem):
        cid = lax.axis_index("c")
        sid = lax.axis_index("s")
        wid = cid * 16 + sid
        pltpu.sync_copy(pk_hbm.at[wid], pk_v)
        _zero_fill(ones_v, EMB)
        row0 = sid * ROWS_PER_TILE
        for r0, nr in _tile_rows(row0):
            pltpu.sync_copy(ones_v.at[pl.ds(0, nr)], acc.at[pl.ds(r0, nr)])
        one = jnp.ones((16,), F32)

        def orow(r, carry):
            for cb in range(EMB // 16):
                ones_v[r, pl.ds(cb * 16, 16)] = one
            return carry

        lax.fori_loop(0, ECHUNK, orow, 0)
        plsc.subcore_barrier()

        def body(j, carry):
            for g in range(8):
                pk = pk_v[j, pl.ds(g * 16, 16)]
                idx[0, pl.ds(g * 16, 16)] = lax.bitwise_and(
                    lax.shift_right_logical(pk, shift), 0x3FFF)
            s = pltpu.async_copy(ones_v, acc.at[idx.at[0]], sem, add=True)
            s.wait()
            return carry

        lax.fori_loop(0, ECHUNKS, body, 0)
        plsc.subcore_barrier()
        for r0, nr in _tile_rows(row0):
            pltpu.sync_copy(acc.at[pl.ds(r0, nr)], out_hbm.at[cid, pl.ds(r0, nr)])

    return count


_count_lo = _make_count(0)
_count_hi = _make_count(14)


@functools.partial(
    pl.kernel,
    out_type=(jax.ShapeDtypeStruct((PPAD, W128), F32),
              jax.ShapeDtypeStruct((PPAD, W128), F32),
              jax.ShapeDtypeStruct((PPAD, W128), F32)),
    mesh=_mesh(),
    compiler_params=pltpu.CompilerParams(use_tc_tiling_on_sc=False),
    scratch_types=[
        pltpu.VMEM((PCHUNKS, ECHUNK), I32),
        pltpu.VMEM((PCHUNKS, ECHUNK), I32),
        pltpu.VMEM((PCHUNKS, ECHUNK), I32),
        pltpu.VMEM((ECHUNK, W128), F32),
        pltpu.VMEM((ECHUNK, W128), F32),
        pltpu.VMEM((ECHUNK, W128), F32),
    ],
)
def _pair_gather(comb_hbm, dis_hbm, ori_hbm, dst_hbm,
                 pa_out, pb_out, ds_out, ori_v, dst_v, row_v,
                 buf_a, buf_b, buf_d):
    cid = lax.axis_index("c")
    sid = lax.axis_index("s")
    wid = cid * 16 + sid
    pltpu.sync_copy(ori_hbm.at[wid], ori_v)
    pltpu.sync_copy(dst_hbm.at[wid], dst_v)
    for j in range(PCHUNKS):
        for g in range(8):
            o16 = ori_v[j, pl.ds(g * 16, 16)]
            d16 = dst_v[j, pl.ds(g * 16, 16)]
            fidx = o16 * N + d16
            row_v[j, pl.ds(g * 16, 16)] = lax.shift_right_logical(fidx, 7)
    for j in range(PCHUNKS):
        base = wid * (PCHUNKS * ECHUNK) + j * ECHUNK
        pltpu.sync_copy(comb_hbm.at[ori_v.at[j]], buf_a)
        pltpu.sync_copy(buf_a, pa_out.at[pl.ds(base, ECHUNK)])
        pltpu.sync_copy(comb_hbm.at[dst_v.at[j]], buf_b)
        pltpu.sync_copy(buf_b, pb_out.at[pl.ds(base, ECHUNK)])
        pltpu.sync_copy(dis_hbm.at[row_v.at[j]], buf_d)
        pltpu.sync_copy(buf_d, ds_out.at[pl.ds(base, ECHUNK)])


BR = 2000
GRID = (N // BR,)


def _dot(a, b):
    return jnp.dot(a, b, preferred_element_type=F32, precision=HI)


def _pad128(z):
    return jnp.concatenate(
        [z, jnp.zeros((z.shape[0], W128 - z.shape[1]), F32)], axis=1)


def _row(width):
    return pl.BlockSpec((BR, width), lambda i: (i, 0))


def _pblk(d=EMB):
    return pl.BlockSpec((2, BR, d), lambda i: (0, i, 0))


def _full(shape):
    nd = len(shape)
    return pl.BlockSpec(shape, lambda i: (0,) * nd)


def _tc1_body(h_ref, xt_ref, nrm_ref, wh_ref, wx_ref, o_ref):
    on = nrm_ref[:, 0:1]
    o_ref[...] = (_dot(h_ref[...] * on, wh_ref[...])
                  + _dot(xt_ref[...] * on, wx_ref[...]))


def _tc_mid_body(p_ref, nrm_ref, b_ref, w_ref, o_ref):
    on = nrm_ref[:, 0:1]
    inn = nrm_ref[:, 1:2]
    a = jax.nn.relu(inn * (p_ref[0, :, :] + p_ref[1, :, :]) + b_ref[...])
    o_ref[...] = _dot(a * on, w_ref[...])


def _tc_gate_body(p_ref, nrm_ref, b_ref, h_ref, xt_ref, wh_ref, wx_ref,
                  z4_ref, zg_ref):
    on = nrm_ref[:, 0:1]
    inn = nrm_ref[:, 1:2]
    ru = jax.nn.sigmoid(inn * (p_ref[0, :, :] + p_ref[1, :, :]) + b_ref[...])
    r = ru[:, :EMB]
    zg = ru[:, EMB:]
    z4_ref[...] = (_dot(h_ref[...] * r * on, wh_ref[...])
                   + _dot(xt_ref[...] * on, wx_ref[...]))
    zg_ref[...] = zg


def _tc_final_body(p_ref, nrm_ref, b_ref, zg_ref, h_ref, wab_ref,
                   h2_ref, proj_ref):
    inn = nrm_ref[:, 1:2]
    hn = jnp.tanh(inn * (p_ref[0, :, :] + p_ref[1, :, :]) + b_ref[...])
    zg = zg_ref[...]
    h2 = zg * h_ref[...] + (1.0 - zg) * hn
    h2_ref[...] = h2
    proj_ref[...] = _dot(h2, wab_ref[...])


def _tc_norms_body(dgo_ref, dgi_ref, nrm_ref):
    od = dgo_ref[0, :, 0:1] + dgo_ref[1, :, 0:1]
    idg = dgi_ref[0, :, 0:1] + dgi_ref[1, :, 0:1]
    on = jnp.where(od > 0, lax.rsqrt(jnp.maximum(od, 1.0)), 0.0)
    inn = jnp.where(idg > 0, lax.rsqrt(jnp.maximum(idg, 1.0)), 0.0)
    nrm_ref[...] = jnp.concatenate([on, inn], axis=1)


PBR = 4096


def _tc_predf_body(pa_ref, pb_ref, ds_ref, col_ref, wt_ref, b_ref, o_ref):
    cols = lax.broadcasted_iota(I32, (1, ECHUNK), 1)
    onehot = (cols == col_ref[...]).astype(F32)
    dv = jnp.sum(ds_ref[...] * onehot, axis=1, keepdims=True)
    s = pa_ref[:, :T] + pb_ref[:, T:2 * T] + dv * wt_ref[...] + b_ref[...]
    o_ref[...] = jnp.tanh(s)


def _call(body, out_shape, *args):
    return pl.pallas_call(body, out_shape=out_shape)(*args)


def _gcall(body, out_shape, out_specs, in_specs, *args):
    return pl.pallas_call(body, out_shape=out_shape, grid=GRID,
                          in_specs=in_specs, out_specs=out_specs)(*args)


def _sds(shape):
    return jax.ShapeDtypeStruct(shape, F32)


def kernel(x, dis, edge_index, train_idx, ru_W_in, ru_b_in, ru_W_hid, ru_b_hid,
           ru_W_out, ru_b_out, rh_W_in, rh_b_in, rh_W_hid, rh_b_hid, rh_W_out,
           rh_b_out, pred_W, pred_b):
    src = edge_index[0].astype(I32)
    dst = edge_index[1].astype(I32)
    npadE = EPAD - E
    fill_g = jnp.arange(npadE, dtype=I32) % N
    fill_d = N + jnp.arange(npadE, dtype=I32) % 16
    srcP = jnp.concatenate([src, fill_g])
    dstP = jnp.concatenate([dst, fill_d])
    pk3 = (srcP | (dstP << 14)).reshape(NTILES, ECHUNKS, ECHUNK)
    srcD = jnp.concatenate([src, fill_d])
    pkd3 = (srcD | (dstP << 14)).reshape(NTILES, ECHUNKS, ECHUNK)

    ori = train_idx[0].astype(I32)
    dstp = train_idx[1].astype(I32)
    pfill = jnp.arange(PPAD - P, dtype=I32) % N
    ori3 = jnp.concatenate([ori, pfill]).reshape(NTILES, PCHUNKS, ECHUNK)
    dst3p = jnp.concatenate([dstp, pfill]).reshape(NTILES, PCHUNKS, ECHUNK)

    feat = x[:, :, :EMB]
    ruW_h, ruW_x = ru_W_in[:EMB], ru_W_in[EMB:]
    rhW_h, rhW_x = rh_W_in[:EMB], rh_W_in[EMB:]
    rub1 = ru_b_in.reshape(1, EMB)
    rub2 = ru_b_hid.reshape(1, EMB)
    rub3 = ru_b_out.reshape(1, 2 * EMB)
    rhb1 = rh_b_in.reshape(1, EMB)
    rhb2 = rh_b_hid.reshape(1, EMB)
    rhb3 = rh_b_out.reshape(1, EMB)
    wab = jnp.concatenate([pred_W[:EMB], pred_W[EMB:2 * EMB]], axis=1)

    dego = _count_lo(pkd3)
    degi = _count_hi(pkd3)
    norms = _gcall(_tc_norms_body, _sds((N, 2)), _row(2),
                   [_pblk(), _pblk()], dego, degi)

    def step(h, xt):
        mid_specs = [_pblk(), _row(2), _full((1, EMB)), _full((EMB, EMB))]
        z1 = _gcall(_tc1_body, _sds((N, EMB)), _row(EMB),
                    [_row(EMB), _row(EMB), _row(2),
                     _full((EMB, EMB)), _full((EMB, EMB))],
                    h, xt, norms, ruW_h, ruW_x)
        p = _seg64(z1, pk3)
        z2 = _gcall(_tc_mid_body, _sds((N, EMB)), _row(EMB), mid_specs,
                    p, norms, rub1, ru_W_hid)
        p = _seg64(z2, pk3)
        z3 = _gcall(_tc_mid_body, _sds((N, W128)), _row(W128),
                    [_pblk(), _row(2), _full((1, EMB)), _full((EMB, W128))],
                    p, norms, rub2, ru_W_out)
        p = _seg128(z3, pk3)
        z4, zg = _gcall(_tc_gate_body, (_sds((N, EMB)), _sds((N, EMB))),
                        (_row(EMB), _row(EMB)),
                        [_pblk(W128), _row(2), _full((1, W128)), _row(EMB),
                         _row(EMB), _full((EMB, EMB)), _full((EMB, EMB))],
                        p, norms, rub3, h, xt, rhW_h, rhW_x)
        p = _seg64(z4, pk3)
        z5 = _gcall(_tc_mid_body, _sds((N, EMB)), _row(EMB), mid_specs,
                    p, norms, rhb1, rh_W_hid)
        p = _seg64(z5, pk3)
        z6 = _gcall(_tc_mid_body, _sds((N, EMB)), _row(EMB), mid_specs,
                    p, norms, rhb2, rh_W_out)
        p = _seg64(z6, pk3)
        h2, proj = _gcall(_tc_final_body, (_sds((N, EMB)), _sds((N, 2))),
                          (_row(EMB), _row(2)),
                          [_pblk(), _row(2), _full((1, EMB)), _row(EMB),
                           _row(EMB), _full((EMB, 2))],
                          p, norms, rhb3, zg, h, wab)
        return h2, proj

    h0 = jnp.zeros((N, EMB), F32)
    _, projs = lax.scan(step, h0, feat)

    paT = projs[:, :, 0].T
    pbT = projs[:, :, 1].T
    comb = jnp.pad(jnp.concatenate([paT, pbT], axis=1),
                   ((0, 0), (0, W128 - 2 * T)))
    pa_g, pb_g, dslab = _pair_gather(comb, dis.reshape(N * N // 128, 128),
                                     ori3, dst3p)
    oriP = jnp.concatenate([ori, pfill])
    dstP = jnp.concatenate([dstp, pfill])
    colid = ((oriP * N + dstP) % 128).astype(I32).reshape(PPAD, 1)
    pr = lambda w: pl.BlockSpec((PBR, w), lambda i: (i, 0))
    out = pl.pallas_call(
        _tc_predf_body, out_shape=_sds((PPAD, T)), grid=(PPAD // PBR,),
        in_specs=[pr(W128), pr(W128), pr(W128), pr(1),
                  _full((1, 1)), _full((1, 1))],
        out_specs=pr(T),
    )(pa_g, pb_g, dslab, colid, pred_W[2 * EMB:].reshape(1, 1),
      pred_b.reshape(1, 1))
    return out[:P]


# speedup vs baseline: 1.4056x; 1.0286x over previous
import functools

import jax
import jax.numpy as jnp
from jax import lax
from jax.experimental import pallas as pl
from jax.experimental.pallas import tpu as pltpu
from jax.experimental.pallas import tpu_sc as plsc

N = 10000
NPAD = 10112
EMB = 64
W128 = 128
T = 24
E = 320000
P = 20000
NTILES = 32
ECHUNK = 128
ECHUNKS = 80
EPAD = NTILES * ECHUNKS * ECHUNK
PCHUNKS = 5
PPAD = NTILES * PCHUNKS * ECHUNK
ROWS_PER_TILE = NPAD // 16
F32 = jnp.float32
I32 = jnp.int32
HI = lax.Precision.HIGHEST


def _mesh():
    return plsc.VectorSubcoreMesh(core_axis_name="c", subcore_axis_name="s",
                                  num_cores=2, num_subcores=16)


def _zero_fill(buf, d):
    zero = jnp.zeros((16,), F32)

    def zrow(r, carry):
        for cb in range(d // 16):
            buf[r, pl.ds(cb * 16, 16)] = zero
        return carry

    lax.fori_loop(0, ECHUNK, zrow, 0)


def _tile_rows(row0):
    out = []
    done = 0
    while done < ROWS_PER_TILE:
        n = min(128, ROWS_PER_TILE - done)
        out.append((row0 + done, n))
        done += n
    return out


def _make_seg_sum(d, nbuf):
  scratch = [
        pltpu.VMEM_SHARED((NPAD, d), F32),
        pltpu.VMEM((ECHUNKS, ECHUNK), I32),
        pltpu.VMEM((nbuf, ECHUNK), I32),
        pltpu.VMEM((nbuf, ECHUNK), I32),
  ]
  scratch += [pltpu.VMEM((ECHUNK, d), F32)] * nbuf
  scratch += [pltpu.SemaphoreType.DMA] * (2 * nbuf)

  @functools.partial(
    pl.kernel,
    out_type=jax.ShapeDtypeStruct((2, NPAD, d), F32),
    mesh=_mesh(),
    compiler_params=pltpu.CompilerParams(use_tc_tiling_on_sc=False),
    scratch_types=scratch,
  )
  def _seg_sum(z_hbm, pk_hbm, out_hbm, acc, pk_v, sidx, didx, *bufs_sems):
    bufs = bufs_sems[:nbuf]
    gsems = bufs_sems[nbuf:2 * nbuf]
    ssems = bufs_sems[2 * nbuf:3 * nbuf]
    cid = lax.axis_index("c")
    sid = lax.axis_index("s")
    wid = cid * 16 + sid
    pltpu.sync_copy(pk_hbm.at[wid], pk_v)
    _zero_fill(bufs[0], d)
    row0 = sid * ROWS_PER_TILE
    for r0, nr in _tile_rows(row0):
        pltpu.sync_copy(bufs[0].at[pl.ds(0, nr)], acc.at[pl.ds(r0, nr)])
    plsc.subcore_barrier()

    def unpack(c, slot):
        for g in range(8):
            pk = pk_v[c, pl.ds(g * 16, 16)]
            sidx[slot, pl.ds(g * 16, 16)] = lax.bitwise_and(pk, 0x3FFF)
            didx[slot, pl.ds(g * 16, 16)] = lax.shift_right_logical(pk, 14)

    def gather(k):
        pltpu.async_copy(z_hbm.at[sidx.at[k]], bufs[k], gsems[k])

    def gather_wait(k):
        pltpu.make_async_copy(z_hbm.at[sidx.at[k]], bufs[k], gsems[k]).wait()

    def scat(k):
        pltpu.async_copy(bufs[k], acc.at[didx.at[k]], ssems[k], add=True)

    def scat_wait(k):
        pltpu.make_async_copy(bufs[k], acc.at[didx.at[k]], ssems[k]).wait()

    for k in range(nbuf):
        unpack(k, k)
        gather(k)

    nloop = ECHUNKS // nbuf
    def body(j, carry):
        for k in range(nbuf):
            gather_wait(k)
            scat(k)
        for k in range(nbuf):
            c = nbuf * j + k + nbuf
            scat_wait(k)

            @pl.when(j < nloop - 1)
            def _():
                unpack(c, k)
                gather(k)
        return carry

    lax.fori_loop(0, nloop, body, 0)
    plsc.subcore_barrier()
    for r0, nr in _tile_rows(row0):
        pltpu.sync_copy(acc.at[pl.ds(r0, nr)], out_hbm.at[cid, pl.ds(r0, nr)])

  return _seg_sum


_seg64 = _make_seg_sum(EMB, nbuf=8)
_seg128 = _make_seg_sum(W128, nbuf=2)


def _make_count(shift):
    @functools.partial(
        pl.kernel,
        out_type=jax.ShapeDtypeStruct((2, NPAD, EMB), F32),
        mesh=_mesh(),
        compiler_params=pltpu.CompilerParams(use_tc_tiling_on_sc=False),
        scratch_types=[
            pltpu.VMEM_SHARED((NPAD, EMB), F32),
            pltpu.VMEM((ECHUNKS, ECHUNK), I32),
            pltpu.VMEM((1, ECHUNK), I32),
            pltpu.VMEM((ECHUNK, EMB), F32),
            pltpu.SemaphoreType.DMA,
        ],
    )
    def count(pk_hbm, out_hbm, acc, pk_v, idx, ones_v, sem):
        cid = lax.axis_index("c")
        sid = lax.axis_index("s")
        wid = cid * 16 + sid
        pltpu.sync_copy(pk_hbm.at[wid], pk_v)
        _zero_fill(ones_v, EMB)
        row0 = sid * ROWS_PER_TILE
        for r0, nr in _tile_rows(row0):
            pltpu.sync_copy(ones_v.at[pl.ds(0, nr)], acc.at[pl.ds(r0, nr)])
        one = jnp.ones((16,), F32)

        def orow(r, carry):
            for cb in range(EMB // 16):
                ones_v[r, pl.ds(cb * 16, 16)] = one
            return carry

        lax.fori_loop(0, ECHUNK, orow, 0)
        plsc.subcore_barrier()

        def body(j, carry):
            for g in range(8):
                pk = pk_v[j, pl.ds(g * 16, 16)]
                idx[0, pl.ds(g * 16, 16)] = lax.bitwise_and(
                    lax.shift_right_logical(pk, shift), 0x3FFF)
            s = pltpu.async_copy(ones_v, acc.at[idx.at[0]], sem, add=True)
            s.wait()
            return carry

        lax.fori_loop(0, ECHUNKS, body, 0)
        plsc.subcore_barrier()
        for r0, nr in _tile_rows(row0):
            pltpu.sync_copy(acc.at[pl.ds(r0, nr)], out_hbm.at[cid, pl.ds(r0, nr)])

    return count


_count_lo = _make_count(0)
_count_hi = _make_count(14)


@functools.partial(
    pl.kernel,
    out_type=(jax.ShapeDtypeStruct((PPAD, W128), F32),
              jax.ShapeDtypeStruct((PPAD, W128), F32),
              jax.ShapeDtypeStruct((PPAD, W128), F32)),
    mesh=_mesh(),
    compiler_params=pltpu.CompilerParams(use_tc_tiling_on_sc=False),
    scratch_types=[
        pltpu.VMEM((PCHUNKS, ECHUNK), I32),
        pltpu.VMEM((PCHUNKS, ECHUNK), I32),
        pltpu.VMEM((PCHUNKS, ECHUNK), I32),
        pltpu.VMEM((ECHUNK, W128), F32),
        pltpu.VMEM((ECHUNK, W128), F32),
        pltpu.VMEM((ECHUNK, W128), F32),
    ],
)
def _pair_gather(comb_hbm, dis_hbm, ori_hbm, dst_hbm,
                 pa_out, pb_out, ds_out, ori_v, dst_v, row_v,
                 buf_a, buf_b, buf_d):
    cid = lax.axis_index("c")
    sid = lax.axis_index("s")
    wid = cid * 16 + sid
    pltpu.sync_copy(ori_hbm.at[wid], ori_v)
    pltpu.sync_copy(dst_hbm.at[wid], dst_v)
    for j in range(PCHUNKS):
        for g in range(8):
            o16 = ori_v[j, pl.ds(g * 16, 16)]
            d16 = dst_v[j, pl.ds(g * 16, 16)]
            fidx = o16 * N + d16
            row_v[j, pl.ds(g * 16, 16)] = lax.shift_right_logical(fidx, 7)
    for j in range(PCHUNKS):
        base = wid * (PCHUNKS * ECHUNK) + j * ECHUNK
        pltpu.sync_copy(comb_hbm.at[ori_v.at[j]], buf_a)
        pltpu.sync_copy(buf_a, pa_out.at[pl.ds(base, ECHUNK)])
        pltpu.sync_copy(comb_hbm.at[dst_v.at[j]], buf_b)
        pltpu.sync_copy(buf_b, pb_out.at[pl.ds(base, ECHUNK)])
        pltpu.sync_copy(dis_hbm.at[row_v.at[j]], buf_d)
        pltpu.sync_copy(buf_d, ds_out.at[pl.ds(base, ECHUNK)])


BR = 2000
GRID = (N // BR,)


def _dot(a, b):
    return jnp.dot(a, b, preferred_element_type=F32, precision=HI)


def _pad128(z):
    return jnp.concatenate(
        [z, jnp.zeros((z.shape[0], W128 - z.shape[1]), F32)], axis=1)


def _row(width):
    return pl.BlockSpec((BR, width), lambda i: (i, 0))


def _pblk(d=EMB):
    return pl.BlockSpec((2, BR, d), lambda i: (0, i, 0))


def _full(shape):
    nd = len(shape)
    return pl.BlockSpec(shape, lambda i: (0,) * nd)


def _tc1_body(h_ref, xt_ref, nrm_ref, wh_ref, wx_ref, o_ref):
    on = nrm_ref[:, 0:1]
    o_ref[...] = (_dot(h_ref[...] * on, wh_ref[...])
                  + _dot(xt_ref[...] * on, wx_ref[...]))


def _tc_mid_body(p_ref, nrm_ref, b_ref, w_ref, o_ref):
    on = nrm_ref[:, 0:1]
    inn = nrm_ref[:, 1:2]
    a = jax.nn.relu(inn * (p_ref[0, :, :] + p_ref[1, :, :]) + b_ref[...])
    o_ref[...] = _dot(a * on, w_ref[...])


def _tc_split_body(p_ref, nrm_ref, b_ref, w_ref, oa_ref, ob_ref):
    on = nrm_ref[:, 0:1]
    inn = nrm_ref[:, 1:2]
    a = jax.nn.relu(inn * (p_ref[0, :, :] + p_ref[1, :, :]) + b_ref[...])
    aon = a * on
    oa_ref[...] = _dot(aon, w_ref[:, :EMB])
    ob_ref[...] = _dot(aon, w_ref[:, EMB:])


def _tc_gate_body(pa_ref, pb_ref, nrm_ref, ba_ref, bb_ref, h_ref, xt_ref,
                  wh_ref, wx_ref, z4_ref, zg_ref):
    on = nrm_ref[:, 0:1]
    inn = nrm_ref[:, 1:2]
    r = jax.nn.sigmoid(inn * (pa_ref[0, :, :] + pa_ref[1, :, :]) + ba_ref[...])
    zg = jax.nn.sigmoid(inn * (pb_ref[0, :, :] + pb_ref[1, :, :]) + bb_ref[...])
    z4_ref[...] = (_dot(h_ref[...] * r * on, wh_ref[...])
                   + _dot(xt_ref[...] * on, wx_ref[...]))
    zg_ref[...] = zg


def _tc_final_body(p_ref, nrm_ref, b_ref, zg_ref, h_ref, wab_ref, xtn_ref,
                   wh_ref, wx_ref, h2_ref, proj_ref, z1n_ref):
    on = nrm_ref[:, 0:1]
    inn = nrm_ref[:, 1:2]
    hn = jnp.tanh(inn * (p_ref[0, :, :] + p_ref[1, :, :]) + b_ref[...])
    zg = zg_ref[...]
    h2 = zg * h_ref[...] + (1.0 - zg) * hn
    h2_ref[...] = h2
    proj_ref[...] = _dot(h2, wab_ref[...])
    z1n_ref[...] = (_dot(h2 * on, wh_ref[...])
                    + _dot(xtn_ref[...] * on, wx_ref[...]))


def _tc_norms_body(dgo_ref, dgi_ref, nrm_ref):
    od = dgo_ref[0, :, 0:1] + dgo_ref[1, :, 0:1]
    idg = dgi_ref[0, :, 0:1] + dgi_ref[1, :, 0:1]
    on = jnp.where(od > 0, lax.rsqrt(jnp.maximum(od, 1.0)), 0.0)
    inn = jnp.where(idg > 0, lax.rsqrt(jnp.maximum(idg, 1.0)), 0.0)
    nrm_ref[...] = jnp.concatenate([on, inn], axis=1)


PBR = 4096


def _tc_predf_body(pa_ref, pb_ref, ds_ref, col_ref, wt_ref, b_ref, o_ref):
    cols = lax.broadcasted_iota(I32, (1, ECHUNK), 1)
    onehot = (cols == col_ref[...]).astype(F32)
    dv = jnp.sum(ds_ref[...] * onehot, axis=1, keepdims=True)
    s = pa_ref[:, :T] + pb_ref[:, T:2 * T] + dv * wt_ref[...] + b_ref[...]
    o_ref[...] = jnp.tanh(s)


def _call(body, out_shape, *args):
    return pl.pallas_call(body, out_shape=out_shape)(*args)


def _gcall(body, out_shape, out_specs, in_specs, *args):
    return pl.pallas_call(body, out_shape=out_shape, grid=GRID,
                          in_specs=in_specs, out_specs=out_specs)(*args)


def _sds(shape):
    return jax.ShapeDtypeStruct(shape, F32)


def kernel(x, dis, edge_index, train_idx, ru_W_in, ru_b_in, ru_W_hid, ru_b_hid,
           ru_W_out, ru_b_out, rh_W_in, rh_b_in, rh_W_hid, rh_b_hid, rh_W_out,
           rh_b_out, pred_W, pred_b):
    src = edge_index[0].astype(I32)
    dst = edge_index[1].astype(I32)
    npadE = EPAD - E
    fill_g = jnp.arange(npadE, dtype=I32) % N
    fill_d = N + jnp.arange(npadE, dtype=I32) % 16
    srcP = jnp.concatenate([src, fill_g])
    dstP = jnp.concatenate([dst, fill_d])
    pk3 = (srcP | (dstP << 14)).reshape(NTILES, ECHUNKS, ECHUNK)
    srcD = jnp.concatenate([src, fill_d])
    pkd3 = (srcD | (dstP << 14)).reshape(NTILES, ECHUNKS, ECHUNK)

    ori = train_idx[0].astype(I32)
    dstp = train_idx[1].astype(I32)
    pfill = jnp.arange(PPAD - P, dtype=I32) % N
    ori3 = jnp.concatenate([ori, pfill]).reshape(NTILES, PCHUNKS, ECHUNK)
    dst3p = jnp.concatenate([dstp, pfill]).reshape(NTILES, PCHUNKS, ECHUNK)

    feat = x[:, :, :EMB]
    ruW_h, ruW_x = ru_W_in[:EMB], ru_W_in[EMB:]
    rhW_h, rhW_x = rh_W_in[:EMB], rh_W_in[EMB:]
    rub1 = ru_b_in.reshape(1, EMB)
    rub2 = ru_b_hid.reshape(1, EMB)
    rub3 = ru_b_out.reshape(1, 2 * EMB)
    rhb1 = rh_b_in.reshape(1, EMB)
    rhb2 = rh_b_hid.reshape(1, EMB)
    rhb3 = rh_b_out.reshape(1, EMB)
    wab = jnp.concatenate([pred_W[:EMB], pred_W[EMB:2 * EMB]], axis=1)

    dego = _count_lo(pkd3)
    degi = _count_hi(pkd3)
    norms = _gcall(_tc_norms_body, _sds((N, 2)), _row(2),
                   [_pblk(), _pblk()], dego, degi)

    mid_specs = [_pblk(), _row(2), _full((1, EMB)), _full((EMB, EMB))]
    rub3a, rub3b = rub3[:, :EMB], rub3[:, EMB:]

    def step(carry, xts):
        h, z1 = carry
        xt, xtn = xts
        p = _seg64(z1, pk3)
        z2 = _gcall(_tc_mid_body, _sds((N, EMB)), _row(EMB), mid_specs,
                    p, norms, rub1, ru_W_hid)
        p = _seg64(z2, pk3)
        z3a, z3b = _gcall(_tc_split_body, (_sds((N, EMB)), _sds((N, EMB))),
                          (_row(EMB), _row(EMB)),
                          [_pblk(), _row(2), _full((1, EMB)), _full((EMB, W128))],
                          p, norms, rub2, ru_W_out)
        pa = _seg64(z3a, pk3)
        pb = _seg64(z3b, pk3)
        z4, zg = _gcall(_tc_gate_body, (_sds((N, EMB)), _sds((N, EMB))),
                        (_row(EMB), _row(EMB)),
                        [_pblk(), _pblk(), _row(2), _full((1, EMB)),
                         _full((1, EMB)), _row(EMB), _row(EMB),
                         _full((EMB, EMB)), _full((EMB, EMB))],
                        pa, pb, norms, rub3a, rub3b, h, xt, rhW_h, rhW_x)
        p = _seg64(z4, pk3)
        z5 = _gcall(_tc_mid_body, _sds((N, EMB)), _row(EMB), mid_specs,
                    p, norms, rhb1, rh_W_hid)
        p = _seg64(z5, pk3)
        z6 = _gcall(_tc_mid_body, _sds((N, EMB)), _row(EMB), mid_specs,
                    p, norms, rhb2, rh_W_out)
        p = _seg64(z6, pk3)
        h2, proj, z1n = _gcall(
            _tc_final_body, (_sds((N, EMB)), _sds((N, 2)), _sds((N, EMB))),
            (_row(EMB), _row(2), _row(EMB)),
            [_pblk(), _row(2), _full((1, EMB)), _row(EMB), _row(EMB),
             _full((EMB, 2)), _row(EMB), _full((EMB, EMB)), _full((EMB, EMB))],
            p, norms, rhb3, zg, h, wab, xtn, ruW_h, ruW_x)
        return (h2, z1n), proj

    h0 = jnp.zeros((N, EMB), F32)
    z1_0 = _gcall(_tc1_body, _sds((N, EMB)), _row(EMB),
                  [_row(EMB), _row(EMB), _row(2),
                   _full((EMB, EMB)), _full((EMB, EMB))],
                  h0, feat[0], norms, ruW_h, ruW_x)
    featn = jnp.concatenate([feat[1:], feat[:1]], axis=0)
    _, projs = lax.scan(step, (h0, z1_0), (feat, featn))

    paT = projs[:, :, 0].T
    pbT = projs[:, :, 1].T
    comb = jnp.pad(jnp.concatenate([paT, pbT], axis=1),
                   ((0, 0), (0, W128 - 2 * T)))
    pa_g, pb_g, dslab = _pair_gather(comb, dis.reshape(N * N // 128, 128),
                                     ori3, dst3p)
    oriP = jnp.concatenate([ori, pfill])
    dstP = jnp.concatenate([dstp, pfill])
    colid = ((oriP * N + dstP) % 128).astype(I32).reshape(PPAD, 1)
    pr = lambda w: pl.BlockSpec((PBR, w), lambda i: (i, 0))
    out = pl.pallas_call(
        _tc_predf_body, out_shape=_sds((PPAD, T)), grid=(PPAD // PBR,),
        in_specs=[pr(W128), pr(W128), pr(W128), pr(1),
                  _full((1, 1)), _full((1, 1))],
        out_specs=pr(T),
    )(pa_g, pb_g, dslab, colid, pred_W[2 * EMB:].reshape(1, 1),
      pred_b.reshape(1, 1))
    return out[:P]
